# Initial kernel scaffold; baseline (speedup 1.0000x reference)
import jax
import jax.numpy as jnp
from jax.experimental import pallas as pl


def kernel(hidden, mem_keys, mem_values, Wq, bq, W1, b1, W2, b2, Wo, bo, filled):
    raise NotImplementedError("write your pallas kernel here")


# speedup vs baseline: 1.0129x; 1.0129x over previous
import math

import jax
import jax.numpy as jnp
from jax.experimental import pallas as pl

B = 128
M = 1024
K = 128
V = 512

BB = 8


def _episodic_kernel(hidden_ref, keys_ref, values_ref, bias_ref,
                     wq_ref, bq_ref, w1h_ref, w1r_ref, b1_ref,
                     w2_ref, b2_ref, wo_ref, bo_ref, out_ref):
    scale = 1.0 / math.sqrt(K)
    h = hidden_ref[...]
    q = jax.lax.dot_general(h, wq_ref[...], (((1,), (1,)), ((), ())),
                            preferred_element_type=jnp.float32) + bq_ref[...]
    scores = jnp.concatenate([
        jax.lax.dot_general(q[j:j + 1], keys_ref[j], (((1,), (1,)), ((), ())),
                            preferred_element_type=jnp.float32)
        for j in range(BB)], axis=0)
    scores = scores * scale + bias_ref[...]
    m = jnp.max(scores, axis=-1, keepdims=True)
    m = jnp.where(jnp.isfinite(m), m, 0.0)
    e = jnp.exp(scores - m)
    s = jnp.sum(e, axis=-1, keepdims=True)
    attn = jnp.where(s > 0.0, e / s, 0.0)
    retrieved = jnp.concatenate([
        jax.lax.dot_general(attn[j:j + 1], values_ref[j], (((1,), (0,)), ((), ())),
                            preferred_element_type=jnp.float32)
        for j in range(BB)], axis=0)
    g = (jax.lax.dot_general(h, w1h_ref[...], (((1,), (1,)), ((), ())),
                             preferred_element_type=jnp.float32)
         + jax.lax.dot_general(retrieved, w1r_ref[...], (((1,), (1,)), ((), ())),
                               preferred_element_type=jnp.float32)
         + b1_ref[...])
    h1 = g * jax.nn.sigmoid(g)
    gate = jax.nn.sigmoid(
        jax.lax.dot_general(h1, w2_ref[...], (((1,), (1,)), ((), ())),
                            preferred_element_type=jnp.float32) + b2_ref[...])
    y = h + gate * retrieved
    out_ref[...] = jax.lax.dot_general(y, wo_ref[...], (((1,), (1,)), ((), ())),
                                       preferred_element_type=jnp.float32) + bo_ref[...]


def kernel(hidden, mem_keys, mem_values, Wq, bq, W1, b1, W2, b2, Wo, bo, filled):
    filled = filled.astype(jnp.int32)
    slot = jax.lax.broadcasted_iota(jnp.int32, (B, M), 1)
    bias = jnp.where(slot < filled[:, None], 0.0, -jnp.inf).astype(jnp.float32)
    W1h = W1[:, :V]
    W1r = W1[:, V:]
    rep2 = lambda i: (0, 0)

    grid = (B // BB,)
    out = pl.pallas_call(
        _episodic_kernel,
        grid=grid,
        in_specs=[
            pl.BlockSpec((BB, V), lambda i: (i, 0)),
            pl.BlockSpec((BB, M, K), lambda i: (i, 0, 0)),
            pl.BlockSpec((BB, M, V), lambda i: (i, 0, 0)),
            pl.BlockSpec((BB, M), lambda i: (i, 0)),
            pl.BlockSpec((K, V), rep2),
            pl.BlockSpec((1, K), rep2),
            pl.BlockSpec((V, V), rep2),
            pl.BlockSpec((V, V), rep2),
            pl.BlockSpec((1, V), rep2),
            pl.BlockSpec((V, V), rep2),
            pl.BlockSpec((1, V), rep2),
            pl.BlockSpec((V, V), rep2),
            pl.BlockSpec((1, V), rep2),
        ],
        out_specs=pl.BlockSpec((BB, V), lambda i: (i, 0)),
        out_shape=jax.ShapeDtypeStruct((B, V), jnp.float32),
    )(hidden, mem_keys, mem_values, bias,
      Wq, bq.reshape(1, K), W1h, W1r, b1.reshape(1, V),
      W2, b2.reshape(1, V), Wo, bo.reshape(1, V))
    return out


# speedup vs baseline: 1.0208x; 1.0078x over previous
import math

import jax
import jax.numpy as jnp
from jax.experimental import pallas as pl

B = 128
M = 1024
K = 128
V = 512

BB = 8


def _episodic_kernel(hidden_ref, keys_ref, values_ref, filled_ref,
                     wq_ref, bq_ref, w1h_ref, w1r_ref, b1_ref,
                     w2_ref, b2_ref, wo_ref, bo_ref, out_ref):
    scale = 1.0 / math.sqrt(K)
    h = hidden_ref[...]
    q = jax.lax.dot_general(h, wq_ref[...], (((1,), (1,)), ((), ())),
                            preferred_element_type=jnp.float32) + bq_ref[...]
    scores = jnp.concatenate([
        jax.lax.dot_general(q[j:j + 1], keys_ref[j], (((1,), (1,)), ((), ())),
                            preferred_element_type=jnp.float32)
        for j in range(BB)], axis=0)
    slot = jax.lax.broadcasted_iota(jnp.int32, (BB, M), 1)
    valid = slot < filled_ref[...]
    scores = jnp.where(valid, scores * scale, -jnp.inf)
    m = jnp.max(scores, axis=-1, keepdims=True)
    m = jnp.where(jnp.isfinite(m), m, 0.0)
    e = jnp.exp(scores - m)
    s = jnp.sum(e, axis=-1, keepdims=True)
    attn = jnp.where(s > 0.0, e / s, 0.0)
    retrieved = jnp.concatenate([
        jax.lax.dot_general(attn[j:j + 1], values_ref[j], (((1,), (0,)), ((), ())),
                            preferred_element_type=jnp.float32)
        for j in range(BB)], axis=0)
    g = (jax.lax.dot_general(h, w1h_ref[...], (((1,), (1,)), ((), ())),
                             preferred_element_type=jnp.float32)
         + jax.lax.dot_general(retrieved, w1r_ref[...], (((1,), (1,)), ((), ())),
                               preferred_element_type=jnp.float32)
         + b1_ref[...])
    h1 = g * jax.nn.sigmoid(g)
    gate = jax.nn.sigmoid(
        jax.lax.dot_general(h1, w2_ref[...], (((1,), (1,)), ((), ())),
                            preferred_element_type=jnp.float32) + b2_ref[...])
    y = h + gate * retrieved
    out_ref[...] = jax.lax.dot_general(y, wo_ref[...], (((1,), (1,)), ((), ())),
                                       preferred_element_type=jnp.float32) + bo_ref[...]


def kernel(hidden, mem_keys, mem_values, Wq, bq, W1, b1, W2, b2, Wo, bo, filled):
    filled2d = filled.astype(jnp.int32).reshape(B, 1)
    W1h = W1[:, :V]
    W1r = W1[:, V:]
    rep2 = lambda i: (0, 0)

    grid = (B // BB,)
    out = pl.pallas_call(
        _episodic_kernel,
        grid=grid,
        in_specs=[
            pl.BlockSpec((BB, V), lambda i: (i, 0)),
            pl.BlockSpec((BB, M, K), lambda i: (i, 0, 0)),
            pl.BlockSpec((BB, M, V), lambda i: (i, 0, 0)),
            pl.BlockSpec((BB, 1), lambda i: (i, 0)),
            pl.BlockSpec((K, V), rep2),
            pl.BlockSpec((1, K), rep2),
            pl.BlockSpec((V, V), rep2),
            pl.BlockSpec((V, V), rep2),
            pl.BlockSpec((1, V), rep2),
            pl.BlockSpec((V, V), rep2),
            pl.BlockSpec((1, V), rep2),
            pl.BlockSpec((V, V), rep2),
            pl.BlockSpec((1, V), rep2),
        ],
        out_specs=pl.BlockSpec((BB, V), lambda i: (i, 0)),
        out_shape=jax.ShapeDtypeStruct((B, V), jnp.float32),
    )(hidden, mem_keys, mem_values, filled2d,
      Wq, bq.reshape(1, K), W1h, W1r, b1.reshape(1, V),
      W2, b2.reshape(1, V), Wo, bo.reshape(1, V))
    return out
